# Initial kernel scaffold; baseline (speedup 1.0000x reference)
import jax
import jax.numpy as jnp
from jax.experimental import pallas as pl


def kernel(x, y, x_jig, memory):
    raise NotImplementedError("write your pallas kernel here")


# speedup vs baseline: 2.9527x; 2.9527x over previous
import jax
import jax.numpy as jnp
from jax import lax
from jax.experimental import pallas as pl
from jax.experimental.pallas import tpu as pltpu
from jax.experimental.pallas import tpu_sc as plsc

_N_DATA = 1000000
_D = 64
_K1 = 4097
_T = 0.07
_M = 0.5
_B = 256
_CHROWS = 128
_KP = 4224
_CPB = _KP // _CHROWS
_NC, _NS, _L = 2, 16, 16
_NW = _NC * _NS
_BPW = _B // _NW
_NCH = _BPW * _CPB


def _sc_body(mem, idx2d, xf, xjf, y, lx, lj, my,
             idx_v, x_v, xj_v, y_v, ym_v, rows0, rows1, lx_v, lj_v,
             gsem0, gsem1, ysem):
    cid = lax.axis_index("c")
    sid = lax.axis_index("s")
    wid = sid * _NC + cid

    pltpu.sync_copy(idx2d.at[pl.ds(wid * _NCH, _NCH)], idx_v)
    pltpu.sync_copy(xf.at[pl.ds(wid * _BPW * _D, _BPW * _D)], x_v)
    pltpu.sync_copy(xjf.at[pl.ds(wid * _BPW * _D, _BPW * _D)], xj_v)
    pltpu.sync_copy(y.at[pl.ds(wid * _BPW, _BPW)], y_v)

    pltpu.make_async_copy(mem.at[y_v], ym_v, ysem).start()
    pltpu.make_async_copy(mem.at[y_v], ym_v, ysem).wait()
    pltpu.sync_copy(ym_v, my.at[pl.ds(wid * _BPW, _BPW)])

    rows = (rows0, rows1)
    gsems = (gsem0, gsem1)

    def start_gather(c, i):
        pltpu.make_async_copy(mem.at[idx_v.at[c]], rows[i], gsems[i]).start()

    def wait_gather(c, i):
        pltpu.make_async_copy(mem.at[idx_v.at[c]], rows[i], gsems[i]).wait()

    rowids = [lax.iota(jnp.int32, _L) + (g * _L) for g in range(8)]
    zero16 = jnp.zeros((_L,), jnp.float32)

    start_gather(0, 0)
    start_gather(1, 1)

    def outer(t, carry):
        for i in range(2):
            c = t * 2 + i
            wait_gather(c, i)
            bl = c // _CPB
            cb = c - bl * _CPB

            base = bl * _D

            def dbody(d, accs):
                sidx = jnp.full((_L,), base + d, jnp.int32)
                dsplat = jnp.full((_L,), d, jnp.int32)
                xd = plsc.load_gather(x_v, [sidx])
                jd = plsc.load_gather(xj_v, [sidx])
                out = []
                for g in range(8):
                    v = plsc.load_gather(rows[i], [rowids[g], dsplat])
                    out.append(accs[2 * g] + v * xd)
                    out.append(accs[2 * g + 1] + v * jd)
                return tuple(out)

            accs = lax.fori_loop(
                0, _D, dbody, tuple(zero16 for _ in range(16)))

            for g in range(8):
                lx_v[pl.ds(cb * _CHROWS + g * _L, _L)] = accs[2 * g]
                lj_v[pl.ds(cb * _CHROWS + g * _L, _L)] = accs[2 * g + 1]

            @pl.when(c + 2 < _NCH)
            def _():
                start_gather(c + 2, i)

            @pl.when(cb == _CPB - 1)
            def _():
                off = (wid * _BPW + bl) * _KP
                pltpu.sync_copy(lx_v, lx.at[pl.ds(off, _KP)])
                pltpu.sync_copy(lj_v, lj.at[pl.ds(off, _KP)])
        return carry

    lax.fori_loop(0, _NCH // 2, outer, 0)


def _sc_logits(memory, idx2d, xf, xjf, y):
    mesh = plsc.VectorSubcoreMesh(
        core_axis_name="c", subcore_axis_name="s", num_cores=_NC,
        num_subcores=_NS)
    f32 = jnp.float32
    kern = pl.kernel(
        _sc_body,
        out_type=(
            jax.ShapeDtypeStruct((_B * _KP,), f32),
            jax.ShapeDtypeStruct((_B * _KP,), f32),
            jax.ShapeDtypeStruct((_B, _D), f32),
        ),
        mesh=mesh,
        compiler_params=pltpu.CompilerParams(
            needs_layout_passes=False, use_tc_tiling_on_sc=False),
        scratch_types=[
            pltpu.VMEM((_NCH, _CHROWS), jnp.int32),
            pltpu.VMEM((_BPW * _D,), f32),
            pltpu.VMEM((_BPW * _D,), f32),
            pltpu.VMEM((_BPW,), jnp.int32),
            pltpu.VMEM((_BPW, _D), f32),
            pltpu.VMEM((_CHROWS, _D), f32),
            pltpu.VMEM((_CHROWS, _D), f32),
            pltpu.VMEM((_KP,), f32),
            pltpu.VMEM((_KP,), f32),
            pltpu.SemaphoreType.DMA,
            pltpu.SemaphoreType.DMA,
            pltpu.SemaphoreType.DMA,
        ],
    )
    return kern(memory, idx2d, xf, xjf, y)


def _loss_body(lx_ref, lj_ref, x_ref, my_ref, loss_ref, wp_ref):
    col = lax.broadcasted_iota(jnp.int32, (_B, _KP), 1)
    valid = col < _K1
    inv_t = jnp.float32(1.0 / _T)

    def ce(ref):
        l = ref[...] * inv_t
        l = jnp.where(valid, l, -jnp.inf)
        m = jnp.max(l, axis=1, keepdims=True)
        s = jnp.sum(jnp.exp(l - m), axis=1, keepdims=True)
        z = jnp.log(s) + m
        return jnp.sum(z - l[:, 0:1]) / _B

    loss = 0.5 * ce(lx_ref) + 0.5 * ce(lj_ref)
    loss_ref[...] = loss.reshape(1, 1)

    wp = my_ref[...] * _M + x_ref[...] * (1.0 - _M)
    wp_ref[...] = wp * lax.rsqrt(jnp.sum(wp * wp, axis=1, keepdims=True))


def _loss_call(lx, lj, x, my):
    return pl.pallas_call(
        _loss_body,
        out_shape=(
            jax.ShapeDtypeStruct((1, 1), jnp.float32),
            jax.ShapeDtypeStruct((_B, _D), jnp.float32),
        ),
    )(lx, lj, x, my)


def _scatter_body(y_ref, wp_blk, mem_any, out_blk):
    del y_ref, mem_any
    out_blk[...] = wp_blk[...]


def _scatter_call(y, wp, memory):
    grid_spec = pltpu.PrefetchScalarGridSpec(
        num_scalar_prefetch=1,
        grid=(_B,),
        in_specs=[
            pl.BlockSpec((1, 1, _D), lambda i, yref: (i, 0, 0)),
            pl.BlockSpec(memory_space=pl.ANY),
        ],
        out_specs=pl.BlockSpec((1, 1, _D), lambda i, yref: (yref[i], 0, 0)),
    )
    out = pl.pallas_call(
        _scatter_body,
        grid_spec=grid_spec,
        out_shape=jax.ShapeDtypeStruct((_N_DATA, 1, _D), jnp.float32),
        input_output_aliases={2: 0},
    )(y, wp.reshape(_B, 1, _D), memory.reshape(_N_DATA, 1, _D))
    return out.reshape(_N_DATA, _D)


def kernel(x, y, x_jig, memory):
    idx_key = jax.random.fold_in(jax.random.key(0), 123)
    idx = jax.random.randint(idx_key, (_B, _K1), 0, _N_DATA)
    y32 = y.astype(idx.dtype)
    idx = idx.at[:, 0].set(y32)
    idx = jnp.pad(idx, ((0, 0), (0, _KP - _K1)))
    idx2d = idx.reshape(_B * _CPB, _CHROWS)

    lxf, ljf, my = _sc_logits(
        memory, idx2d, x.reshape(-1), x_jig.reshape(-1), y32)
    lx = lxf.reshape(_B, _KP)
    lj = ljf.reshape(_B, _KP)

    loss11, wp = _loss_call(lx, lj, x, my)
    new_memory = _scatter_call(y32, wp, memory)
    return loss11.reshape(()), new_memory


# speedup vs baseline: 3.7991x; 1.2866x over previous
import jax
import jax.numpy as jnp
from jax import lax
from jax.experimental import pallas as pl
from jax.experimental.pallas import tpu as pltpu
from jax.experimental.pallas import tpu_sc as plsc

_N_DATA = 1000000
_D = 64
_K1 = 4097
_T = 0.07
_M = 0.5
_B = 256
_CHROWS = 128
_KP = 4224
_CPB = _KP // _CHROWS
_NC, _NS, _L = 2, 16, 16
_NW = _NC * _NS
_BPW = _B // _NW
_NCH = _BPW * _CPB


def _sc_body(mem, idx2d, xf, xjf, y, lx, lj, my,
             idx_v, x_v, xj_v, y_v, ym_v, rows0, rows1, lx_v, lj_v,
             gsem0, gsem1, ysem):
    cid = lax.axis_index("c")
    sid = lax.axis_index("s")
    wid = sid * _NC + cid

    pltpu.sync_copy(idx2d.at[pl.ds(wid * _NCH, _NCH)], idx_v)
    pltpu.sync_copy(xf.at[pl.ds(wid * _BPW * _D, _BPW * _D)], x_v)
    pltpu.sync_copy(xjf.at[pl.ds(wid * _BPW * _D, _BPW * _D)], xj_v)
    pltpu.sync_copy(y.at[pl.ds(wid * _BPW, _BPW)], y_v)

    pltpu.make_async_copy(mem.at[y_v], ym_v, ysem).start()
    pltpu.make_async_copy(mem.at[y_v], ym_v, ysem).wait()
    pltpu.sync_copy(ym_v, my.at[pl.ds(wid * _BPW, _BPW)])

    rows = (rows0, rows1)
    gsems = (gsem0, gsem1)

    def start_gather(c, i):
        pltpu.make_async_copy(mem.at[idx_v.at[c]], rows[i], gsems[i]).start()

    def wait_gather(c, i):
        pltpu.make_async_copy(mem.at[idx_v.at[c]], rows[i], gsems[i]).wait()

    lane15 = lax.iota(jnp.int32, _L) == (_L - 1)
    unroll = 8

    start_gather(0, 0)
    start_gather(1, 1)

    def outer(t, carry):
        for i in range(2):
            c = t * 2 + i
            wait_gather(c, i)
            bl = c // _CPB
            cb = c - bl * _CPB

            base = bl * _D
            xsegs = [x_v[pl.ds(base + q * _L, _L)] for q in range(4)]
            jsegs = [xj_v[pl.ds(base + q * _L, _L)] for q in range(4)]
            out0 = cb * _CHROWS

            def rgroup(t2, _):
                j0 = t2 * unroll
                for u in range(unroll):
                    j = j0 + u
                    r = [rows[i][j, pl.ds(q * _L, _L)] for q in range(4)]
                    sx = ((r[0] * xsegs[0] + r[1] * xsegs[1])
                          + (r[2] * xsegs[2] + r[3] * xsegs[3]))
                    sj = ((r[0] * jsegs[0] + r[1] * jsegs[1])
                          + (r[2] * jsegs[2] + r[3] * jsegs[3]))
                    cx = plsc.cumsum(sx)
                    cj = plsc.cumsum(sj)
                    pos = jnp.full((_L,), out0 + j, jnp.int32)
                    plsc.store_scatter(lx_v, [pos], cx, mask=lane15)
                    plsc.store_scatter(lj_v, [pos], cj, mask=lane15)
                return 0

            lax.fori_loop(0, _CHROWS // unroll, rgroup, 0)

            @pl.when(c + 2 < _NCH)
            def _():
                start_gather(c + 2, i)

            @pl.when(cb == _CPB - 1)
            def _():
                off = (wid * _BPW + bl) * _KP
                pltpu.sync_copy(lx_v, lx.at[pl.ds(off, _KP)])
                pltpu.sync_copy(lj_v, lj.at[pl.ds(off, _KP)])
        return carry

    lax.fori_loop(0, _NCH // 2, outer, 0)


def _sc_logits(memory, idx2d, xf, xjf, y):
    mesh = plsc.VectorSubcoreMesh(
        core_axis_name="c", subcore_axis_name="s", num_cores=_NC,
        num_subcores=_NS)
    f32 = jnp.float32
    kern = pl.kernel(
        _sc_body,
        out_type=(
            jax.ShapeDtypeStruct((_B * _KP,), f32),
            jax.ShapeDtypeStruct((_B * _KP,), f32),
            jax.ShapeDtypeStruct((_B, _D), f32),
        ),
        mesh=mesh,
        compiler_params=pltpu.CompilerParams(
            needs_layout_passes=False, use_tc_tiling_on_sc=False),
        scratch_types=[
            pltpu.VMEM((_NCH, _CHROWS), jnp.int32),
            pltpu.VMEM((_BPW * _D,), f32),
            pltpu.VMEM((_BPW * _D,), f32),
            pltpu.VMEM((_BPW,), jnp.int32),
            pltpu.VMEM((_BPW, _D), f32),
            pltpu.VMEM((_CHROWS, _D), f32),
            pltpu.VMEM((_CHROWS, _D), f32),
            pltpu.VMEM((_KP,), f32),
            pltpu.VMEM((_KP,), f32),
            pltpu.SemaphoreType.DMA,
            pltpu.SemaphoreType.DMA,
            pltpu.SemaphoreType.DMA,
        ],
    )
    return kern(memory, idx2d, xf, xjf, y)


def _loss_body(lx_ref, lj_ref, x_ref, my_ref, loss_ref, wp_ref):
    col = lax.broadcasted_iota(jnp.int32, (_B, _KP), 1)
    valid = col < _K1
    inv_t = jnp.float32(1.0 / _T)

    def ce(ref):
        l = ref[...] * inv_t
        l = jnp.where(valid, l, -jnp.inf)
        m = jnp.max(l, axis=1, keepdims=True)
        s = jnp.sum(jnp.exp(l - m), axis=1, keepdims=True)
        z = jnp.log(s) + m
        return jnp.sum(z - l[:, 0:1]) / _B

    loss = 0.5 * ce(lx_ref) + 0.5 * ce(lj_ref)
    loss_ref[...] = loss.reshape(1, 1)

    wp = my_ref[...] * _M + x_ref[...] * (1.0 - _M)
    wp_ref[...] = wp * lax.rsqrt(jnp.sum(wp * wp, axis=1, keepdims=True))


def _loss_call(lx, lj, x, my):
    return pl.pallas_call(
        _loss_body,
        out_shape=(
            jax.ShapeDtypeStruct((1, 1), jnp.float32),
            jax.ShapeDtypeStruct((_B, _D), jnp.float32),
        ),
    )(lx, lj, x, my)


def _scatter_body(y_ref, wp_blk, mem_any, out_blk):
    del y_ref, mem_any
    out_blk[...] = wp_blk[...]


def _scatter_call(y, wp, memory):
    grid_spec = pltpu.PrefetchScalarGridSpec(
        num_scalar_prefetch=1,
        grid=(_B,),
        in_specs=[
            pl.BlockSpec((1, 1, _D), lambda i, yref: (i, 0, 0)),
            pl.BlockSpec(memory_space=pl.ANY),
        ],
        out_specs=pl.BlockSpec((1, 1, _D), lambda i, yref: (yref[i], 0, 0)),
    )
    out = pl.pallas_call(
        _scatter_body,
        grid_spec=grid_spec,
        out_shape=jax.ShapeDtypeStruct((_N_DATA, 1, _D), jnp.float32),
        input_output_aliases={2: 0},
    )(y, wp.reshape(_B, 1, _D), memory.reshape(_N_DATA, 1, _D))
    return out.reshape(_N_DATA, _D)


def kernel(x, y, x_jig, memory):
    idx_key = jax.random.fold_in(jax.random.key(0), 123)
    idx = jax.random.randint(idx_key, (_B, _K1), 0, _N_DATA)
    y32 = y.astype(idx.dtype)
    idx = idx.at[:, 0].set(y32)
    idx = jnp.pad(idx, ((0, 0), (0, _KP - _K1)))
    idx2d = idx.reshape(_B * _CPB, _CHROWS)

    lxf, ljf, my = _sc_logits(
        memory, idx2d, x.reshape(-1), x_jig.reshape(-1), y32)
    lx = lxf.reshape(_B, _KP)
    lj = ljf.reshape(_B, _KP)

    loss11, wp = _loss_call(lx, lj, x, my)
    new_memory = _scatter_call(y32, wp, memory)
    return loss11.reshape(()), new_memory


# speedup vs baseline: 4.2436x; 1.1170x over previous
import jax
import jax.numpy as jnp
from jax import lax
from jax.experimental import pallas as pl
from jax.experimental.pallas import tpu as pltpu
from jax.experimental.pallas import tpu_sc as plsc

_N_DATA = 1000000
_D = 64
_K1 = 4097
_T = 0.07
_M = 0.5
_B = 256
_CHROWS = 128
_KP = 4224
_CPB = _KP // _CHROWS
_NC, _NS, _L = 2, 16, 16
_NW = _NC * _NS
_BPW = _B // _NW
_NCH = _BPW * _CPB


def _sc_body(mem, idx2d, xf, xjf, y, lx, lj, my,
             idx_v, x_v, xj_v, y_v, ym_v, rows0, rows1, lx_v, lj_v,
             gsem0, gsem1, ysem):
    cid = lax.axis_index("c")
    sid = lax.axis_index("s")
    wid = sid * _NC + cid

    pltpu.sync_copy(idx2d.at[pl.ds(wid * _NCH, _NCH)], idx_v)
    pltpu.sync_copy(xf.at[pl.ds(wid * _BPW * _D, _BPW * _D)], x_v)
    pltpu.sync_copy(xjf.at[pl.ds(wid * _BPW * _D, _BPW * _D)], xj_v)
    pltpu.sync_copy(y.at[pl.ds(wid * _BPW, _BPW)], y_v)

    pltpu.make_async_copy(mem.at[y_v], ym_v, ysem).start()
    pltpu.make_async_copy(mem.at[y_v], ym_v, ysem).wait()
    pltpu.sync_copy(ym_v, my.at[pl.ds(wid * _BPW, _BPW)])

    rows = (rows0, rows1)
    gsems = (gsem0, gsem1)

    def start_gather(c, i):
        pltpu.make_async_copy(mem.at[idx_v.at[c]], rows[i], gsems[i]).start()

    def wait_gather(c, i):
        pltpu.make_async_copy(mem.at[idx_v.at[c]], rows[i], gsems[i]).wait()

    lane15 = lax.iota(jnp.int32, _L) == (_L - 1)
    unroll = 8

    start_gather(0, 0)
    start_gather(1, 1)

    def outer(t, carry):
        for i in range(2):
            c = t * 2 + i
            wait_gather(c, i)
            bl = c // _CPB
            cb = c - bl * _CPB

            base = bl * _D
            xsegs = [x_v[pl.ds(base + q * _L, _L)] for q in range(4)]
            jsegs = [xj_v[pl.ds(base + q * _L, _L)] for q in range(4)]
            out0 = cb * _CHROWS

            @plsc.parallel_loop(0, _CHROWS, unroll=unroll)
            def _(j):
                r = [rows[i][j, pl.ds(q * _L, _L)] for q in range(4)]
                sx = ((r[0] * xsegs[0] + r[1] * xsegs[1])
                      + (r[2] * xsegs[2] + r[3] * xsegs[3]))
                sj = ((r[0] * jsegs[0] + r[1] * jsegs[1])
                      + (r[2] * jsegs[2] + r[3] * jsegs[3]))
                cx = plsc.cumsum(sx)
                cj = plsc.cumsum(sj)
                pos = jnp.full((_L,), out0 + j, jnp.int32)
                plsc.store_scatter(lx_v, [pos], cx, mask=lane15)
                plsc.store_scatter(lj_v, [pos], cj, mask=lane15)

            @pl.when(c + 2 < _NCH)
            def _():
                start_gather(c + 2, i)

            @pl.when(cb == _CPB - 1)
            def _():
                off = (wid * _BPW + bl) * _KP
                pltpu.sync_copy(lx_v, lx.at[pl.ds(off, _KP)])
                pltpu.sync_copy(lj_v, lj.at[pl.ds(off, _KP)])
        return carry

    lax.fori_loop(0, _NCH // 2, outer, 0)


def _sc_logits(memory, idx2d, xf, xjf, y):
    mesh = plsc.VectorSubcoreMesh(
        core_axis_name="c", subcore_axis_name="s", num_cores=_NC,
        num_subcores=_NS)
    f32 = jnp.float32
    kern = pl.kernel(
        _sc_body,
        out_type=(
            jax.ShapeDtypeStruct((_B * _KP,), f32),
            jax.ShapeDtypeStruct((_B * _KP,), f32),
            jax.ShapeDtypeStruct((_B, _D), f32),
        ),
        mesh=mesh,
        compiler_params=pltpu.CompilerParams(
            needs_layout_passes=False, use_tc_tiling_on_sc=False),
        scratch_types=[
            pltpu.VMEM((_NCH, _CHROWS), jnp.int32),
            pltpu.VMEM((_BPW * _D,), f32),
            pltpu.VMEM((_BPW * _D,), f32),
            pltpu.VMEM((_BPW,), jnp.int32),
            pltpu.VMEM((_BPW, _D), f32),
            pltpu.VMEM((_CHROWS, _D), f32),
            pltpu.VMEM((_CHROWS, _D), f32),
            pltpu.VMEM((_KP,), f32),
            pltpu.VMEM((_KP,), f32),
            pltpu.SemaphoreType.DMA,
            pltpu.SemaphoreType.DMA,
            pltpu.SemaphoreType.DMA,
        ],
    )
    return kern(memory, idx2d, xf, xjf, y)


def _loss_body(lx_ref, lj_ref, x_ref, my_ref, loss_ref, wp_ref):
    col = lax.broadcasted_iota(jnp.int32, (_B, _KP), 1)
    valid = col < _K1
    inv_t = jnp.float32(1.0 / _T)

    def ce(ref):
        l = ref[...] * inv_t
        l = jnp.where(valid, l, -jnp.inf)
        m = jnp.max(l, axis=1, keepdims=True)
        s = jnp.sum(jnp.exp(l - m), axis=1, keepdims=True)
        z = jnp.log(s) + m
        return jnp.sum(z - l[:, 0:1]) / _B

    loss = 0.5 * ce(lx_ref) + 0.5 * ce(lj_ref)
    loss_ref[...] = loss.reshape(1, 1)

    wp = my_ref[...] * _M + x_ref[...] * (1.0 - _M)
    wp_ref[...] = wp * lax.rsqrt(jnp.sum(wp * wp, axis=1, keepdims=True))


def _loss_call(lx, lj, x, my):
    return pl.pallas_call(
        _loss_body,
        out_shape=(
            jax.ShapeDtypeStruct((1, 1), jnp.float32),
            jax.ShapeDtypeStruct((_B, _D), jnp.float32),
        ),
    )(lx, lj, x, my)


def _scatter_body(y_pref, yv_ref, wpt_ref, mem_blk, out_blk):
    i = pl.program_id(0)
    blk = y_pref[i] // 128
    yv = yv_ref[0, :]
    colof = jnp.where(yv // 128 == blk, yv % 128, -1)
    jgrid = lax.broadcasted_iota(jnp.int32, (_B, 128), 0)
    cgrid = lax.broadcasted_iota(jnp.int32, (_B, 128), 1)
    hitjc = colof[:, None] == cgrid
    jmax = jnp.max(jnp.where(hitjc, jgrid, -1), axis=0)
    selected = hitjc & (jgrid == jmax[None, :])
    upd = jnp.dot(wpt_ref[...], selected.astype(jnp.float32),
                  preferred_element_type=jnp.float32)
    mask = (jmax >= 0)[None, :]
    out_blk[...] = jnp.where(mask, upd, mem_blk[...])


def _scatter_call(y_sorted, wpt, memory_t):
    grid_spec = pltpu.PrefetchScalarGridSpec(
        num_scalar_prefetch=1,
        grid=(_B,),
        in_specs=[
            pl.BlockSpec((1, _B), lambda i, yref: (0, 0)),
            pl.BlockSpec((_D, _B), lambda i, yref: (0, 0)),
            pl.BlockSpec((_D, 128), lambda i, yref: (0, yref[i] // 128)),
        ],
        out_specs=pl.BlockSpec((_D, 128), lambda i, yref: (0, yref[i] // 128)),
    )
    out = pl.pallas_call(
        _scatter_body,
        grid_spec=grid_spec,
        out_shape=jax.ShapeDtypeStruct((_D, _N_DATA), jnp.float32),
        input_output_aliases={3: 0},
    )(y_sorted, y_sorted.reshape(1, _B), wpt, memory_t)
    return out.T


def kernel(x, y, x_jig, memory):
    idx_key = jax.random.fold_in(jax.random.key(0), 123)
    idx = jax.random.randint(idx_key, (_B, _K1), 0, _N_DATA)
    y32 = y.astype(idx.dtype)
    idx = idx.at[:, 0].set(y32)
    idx = jnp.pad(idx, ((0, 0), (0, _KP - _K1)))
    idx2d = idx.reshape(_B * _CPB, _CHROWS)

    lxf, ljf, my = _sc_logits(
        memory, idx2d, x.reshape(-1), x_jig.reshape(-1), y32)
    lx = lxf.reshape(_B, _KP)
    lj = ljf.reshape(_B, _KP)

    loss11, wp = _loss_call(lx, lj, x, my)

    perm = jnp.argsort(y32)
    new_memory = _scatter_call(y32[perm], wp[perm].T, memory.T)
    return loss11.reshape(()), new_memory


# speedup vs baseline: 4.2944x; 1.0120x over previous
import jax
import jax.numpy as jnp
from jax import lax
from jax.experimental import pallas as pl
from jax.experimental.pallas import tpu as pltpu
from jax.experimental.pallas import tpu_sc as plsc

_N_DATA = 1000000
_D = 64
_K1 = 4097
_T = 0.07
_M = 0.5
_B = 256
_CHROWS = 128
_KP = 4224
_CPB = _KP // _CHROWS
_NC, _NS, _L = 2, 16, 16
_NW = _NC * _NS
_BPW = _B // _NW
_NCH = _BPW * _CPB


def _sc_body(mem, idx2d, xf, xjf, y, lx, lj, my,
             idx_v, x_v, xj_v, y_v, ym_v,
             rows0, rows1, rows2, rows3, rows4, rows5, lx_v, lj_v,
             gsem0, gsem1, gsem2, gsem3, gsem4, gsem5, osem, ysem):
    cid = lax.axis_index("c")
    sid = lax.axis_index("s")
    wid = sid * _NC + cid

    pltpu.sync_copy(idx2d.at[pl.ds(wid * _NCH, _NCH)], idx_v)
    pltpu.sync_copy(xf.at[pl.ds(wid * _BPW * _D, _BPW * _D)], x_v)
    pltpu.sync_copy(xjf.at[pl.ds(wid * _BPW * _D, _BPW * _D)], xj_v)
    pltpu.sync_copy(y.at[pl.ds(wid * _BPW, _BPW)], y_v)

    pltpu.make_async_copy(mem.at[y_v], ym_v, ysem).start()
    pltpu.make_async_copy(mem.at[y_v], ym_v, ysem).wait()
    pltpu.sync_copy(ym_v, my.at[pl.ds(wid * _BPW, _BPW)])

    nbuf = 6
    rows = (rows0, rows1, rows2, rows3, rows4, rows5)
    gsems = (gsem0, gsem1, gsem2, gsem3, gsem4, gsem5)

    def start_gather(c, i):
        pltpu.make_async_copy(mem.at[idx_v.at[c]], rows[i], gsems[i]).start()

    def wait_gather(c, i):
        pltpu.make_async_copy(mem.at[idx_v.at[c]], rows[i], gsems[i]).wait()

    def out_desc(slot, src_v, dst, b):
        return pltpu.make_async_copy(
            src_v.at[pl.ds(slot * _KP, _KP)],
            dst.at[pl.ds((wid * _BPW + b) * _KP, _KP)], osem)

    lane15 = lax.iota(jnp.int32, _L) == (_L - 1)
    unroll = 8

    for i in range(nbuf - 1):
        start_gather(i, i)

    def outer(t, carry):
        for i in range(nbuf):
            c = t * nbuf + i

            @pl.when(c + nbuf - 1 < _NCH)
            def _():
                start_gather(c + nbuf - 1, (i + nbuf - 1) % nbuf)

            wait_gather(c, i)
            bl = c // _CPB
            cb = c - bl * _CPB
            slot = lax.rem(bl, 2)

            @pl.when((cb == 0) & (bl >= 2))
            def _():
                out_desc(slot, lx_v, lx, bl - 2).wait()
                out_desc(slot, lj_v, lj, bl - 2).wait()

            base = bl * _D
            xsegs = [x_v[pl.ds(base + q * _L, _L)] for q in range(4)]
            jsegs = [xj_v[pl.ds(base + q * _L, _L)] for q in range(4)]
            out0 = slot * _KP + cb * _CHROWS

            @plsc.parallel_loop(0, _CHROWS, unroll=unroll)
            def _(j):
                r = [rows[i][j, pl.ds(q * _L, _L)] for q in range(4)]
                sx = ((r[0] * xsegs[0] + r[1] * xsegs[1])
                      + (r[2] * xsegs[2] + r[3] * xsegs[3]))
                sj = ((r[0] * jsegs[0] + r[1] * jsegs[1])
                      + (r[2] * jsegs[2] + r[3] * jsegs[3]))
                cx = plsc.cumsum(sx)
                cj = plsc.cumsum(sj)
                pos = jnp.full((_L,), out0 + j, jnp.int32)
                plsc.store_scatter(lx_v, [pos], cx, mask=lane15)
                plsc.store_scatter(lj_v, [pos], cj, mask=lane15)

            @pl.when(cb == _CPB - 1)
            def _():
                out_desc(slot, lx_v, lx, bl).start()
                out_desc(slot, lj_v, lj, bl).start()
        return carry

    lax.fori_loop(0, _NCH // nbuf, outer, 0)

    for b in (_BPW - 2, _BPW - 1):
        out_desc(b % 2, lx_v, lx, b).wait()
        out_desc(b % 2, lj_v, lj, b).wait()


def _sc_logits(memory, idx2d, xf, xjf, y):
    mesh = plsc.VectorSubcoreMesh(
        core_axis_name="c", subcore_axis_name="s", num_cores=_NC,
        num_subcores=_NS)
    f32 = jnp.float32
    kern = pl.kernel(
        _sc_body,
        out_type=(
            jax.ShapeDtypeStruct((_B * _KP,), f32),
            jax.ShapeDtypeStruct((_B * _KP,), f32),
            jax.ShapeDtypeStruct((_B, _D), f32),
        ),
        mesh=mesh,
        compiler_params=pltpu.CompilerParams(
            needs_layout_passes=False, use_tc_tiling_on_sc=False),
        scratch_types=[
            pltpu.VMEM((_NCH, _CHROWS), jnp.int32),
            pltpu.VMEM((_BPW * _D,), f32),
            pltpu.VMEM((_BPW * _D,), f32),
            pltpu.VMEM((_BPW,), jnp.int32),
            pltpu.VMEM((_BPW, _D), f32),
            pltpu.VMEM((_CHROWS, _D), f32),
            pltpu.VMEM((_CHROWS, _D), f32),
            pltpu.VMEM((_CHROWS, _D), f32),
            pltpu.VMEM((_CHROWS, _D), f32),
            pltpu.VMEM((_CHROWS, _D), f32),
            pltpu.VMEM((_CHROWS, _D), f32),
            pltpu.VMEM((2 * _KP,), f32),
            pltpu.VMEM((2 * _KP,), f32),
            pltpu.SemaphoreType.DMA,
            pltpu.SemaphoreType.DMA,
            pltpu.SemaphoreType.DMA,
            pltpu.SemaphoreType.DMA,
            pltpu.SemaphoreType.DMA,
            pltpu.SemaphoreType.DMA,
            pltpu.SemaphoreType.DMA,
            pltpu.SemaphoreType.DMA,
        ],
    )
    return kern(memory, idx2d, xf, xjf, y)


def _loss_body(lx_ref, lj_ref, x_ref, my_ref, loss_ref, wp_ref):
    col = lax.broadcasted_iota(jnp.int32, (_B, _KP), 1)
    valid = col < _K1
    inv_t = jnp.float32(1.0 / _T)

    def ce(ref):
        l = ref[...] * inv_t
        l = jnp.where(valid, l, -jnp.inf)
        m = jnp.max(l, axis=1, keepdims=True)
        s = jnp.sum(jnp.exp(l - m), axis=1, keepdims=True)
        z = jnp.log(s) + m
        return jnp.sum(z - l[:, 0:1]) / _B

    loss = 0.5 * ce(lx_ref) + 0.5 * ce(lj_ref)
    loss_ref[...] = loss.reshape(1, 1)

    wp = my_ref[...] * _M + x_ref[...] * (1.0 - _M)
    wp_ref[...] = wp * lax.rsqrt(jnp.sum(wp * wp, axis=1, keepdims=True))


def _loss_call(lx, lj, x, my):
    return pl.pallas_call(
        _loss_body,
        out_shape=(
            jax.ShapeDtypeStruct((1, 1), jnp.float32),
            jax.ShapeDtypeStruct((_B, _D), jnp.float32),
        ),
    )(lx, lj, x, my)


def _scatter_body(y_pref, yv_ref, wpt_ref, mem_blk, out_blk):
    i = pl.program_id(0)
    blk = y_pref[i] // 128
    yv = yv_ref[0, :]
    colof = jnp.where(yv // 128 == blk, yv % 128, -1)
    jgrid = lax.broadcasted_iota(jnp.int32, (_B, 128), 0)
    cgrid = lax.broadcasted_iota(jnp.int32, (_B, 128), 1)
    hitjc = colof[:, None] == cgrid
    jmax = jnp.max(jnp.where(hitjc, jgrid, -1), axis=0)
    selected = hitjc & (jgrid == jmax[None, :])
    upd = jnp.dot(wpt_ref[...], selected.astype(jnp.float32),
                  preferred_element_type=jnp.float32)
    mask = (jmax >= 0)[None, :]
    out_blk[...] = jnp.where(mask, upd, mem_blk[...])


def _scatter_call(y_sorted, wpt, memory_t):
    grid_spec = pltpu.PrefetchScalarGridSpec(
        num_scalar_prefetch=1,
        grid=(_B,),
        in_specs=[
            pl.BlockSpec((1, _B), lambda i, yref: (0, 0)),
            pl.BlockSpec((_D, _B), lambda i, yref: (0, 0)),
            pl.BlockSpec((_D, 128), lambda i, yref: (0, yref[i] // 128)),
        ],
        out_specs=pl.BlockSpec((_D, 128), lambda i, yref: (0, yref[i] // 128)),
    )
    out = pl.pallas_call(
        _scatter_body,
        grid_spec=grid_spec,
        out_shape=jax.ShapeDtypeStruct((_D, _N_DATA), jnp.float32),
        input_output_aliases={3: 0},
    )(y_sorted, y_sorted.reshape(1, _B), wpt, memory_t)
    return out.T


def kernel(x, y, x_jig, memory):
    idx_key = jax.random.fold_in(jax.random.key(0), 123)
    idx = jax.random.randint(idx_key, (_B, _K1), 0, _N_DATA)
    y32 = y.astype(idx.dtype)
    idx = idx.at[:, 0].set(y32)
    idx = jnp.pad(idx, ((0, 0), (0, _KP - _K1)))
    idx2d = idx.reshape(_B * _CPB, _CHROWS)

    lxf, ljf, my = _sc_logits(
        memory, idx2d, x.reshape(-1), x_jig.reshape(-1), y32)
    lx = lxf.reshape(_B, _KP)
    lj = ljf.reshape(_B, _KP)

    loss11, wp = _loss_call(lx, lj, x, my)

    perm = jnp.argsort(y32)
    new_memory = _scatter_call(y32[perm], wp[perm].T, memory.T)
    return loss11.reshape(()), new_memory
